# Initial kernel scaffold; baseline (speedup 1.0000x reference)
import jax
import jax.numpy as jnp
from jax.experimental import pallas as pl


def kernel(xyz1, xyz2):
    raise NotImplementedError("write your pallas kernel here")


# speedup vs baseline: 27.0235x; 27.0235x over previous
import functools

import jax
import jax.numpy as jnp
from jax import lax
from jax.experimental import pallas as pl

K = 8
W = 0.1
N = 2048
B = 8


def _rowmin_kth(E, k):
    m = jnp.full((E.shape[0], 1), -jnp.inf, dtype=E.dtype)
    for _ in range(k):
        cand = jnp.where(E > m, E, jnp.inf)
        m = jnp.min(cand, axis=1, keepdims=True)
    return m


def _augment(x):
    x0 = x[:, 0:1]
    x1 = x[:, 1:2]
    x2 = x[:, 2:3]
    aa = x0 * x0 + x1 * x1 + x2 * x2
    ones = jnp.ones_like(x0)
    lh = jnp.concatenate([x, ones], axis=1)
    rh = jnp.concatenate([-2.0 * x, aa], axis=1)
    E = lax.dot_general(lh, rh, (((1,), (1,)), ((), ())),
                        preferred_element_type=jnp.float32)
    thr = _rowmin_kth(E, K)
    Msel = (E <= thr).astype(jnp.float32)
    F = jnp.concatenate(
        [x,
         x0 * x0, x0 * x1, x0 * x2,
         x1 * x0, x1 * x1, x1 * x2,
         x2 * x0, x2 * x1, x2 * x2], axis=1)
    S = lax.dot_general(Msel, F, (((1,), (0,)), ((), ())),
                        preferred_element_type=jnp.float32)
    xs = (x0, x1, x2)
    cols = []
    for a in range(3):
        for b in range(3):
            c = (S[:, 3 + 3 * a + b:4 + 3 * a + b]
                 - S[:, a:a + 1] * xs[b]
                 - xs[a] * S[:, b:b + 1]
                 + float(K) * xs[a] * xs[b])
            cols.append((W / float(K)) * c)
    return jnp.concatenate([x] + cols, axis=1)


def _step(x1_ref, x2_ref, d1_ref, d2_ref):
    x1 = x1_ref[0]
    x2 = x2_ref[0]
    s1 = _augment(x1)
    s2 = _augment(x2)
    sa = jnp.sum(s1 * s1, axis=1, keepdims=True)
    sb = jnp.sum(s2 * s2, axis=1, keepdims=True)
    ones = jnp.ones_like(sa)
    lh = jnp.concatenate([s1, ones], axis=1)
    rh = jnp.concatenate([-2.0 * s2, sb], axis=1)
    E12 = lax.dot_general(lh, rh, (((1,), (1,)), ((), ())),
                          preferred_element_type=jnp.float32)
    D12 = E12 + sa
    d1_ref[0] = jnp.min(D12, axis=1, keepdims=True)
    d2_ref[0] = jnp.min(D12, axis=0, keepdims=True)


@jax.jit
def kernel(xyz1, xyz2):
    d1, d2 = pl.pallas_call(
        _step,
        grid=(B,),
        in_specs=[
            pl.BlockSpec((1, N, 3), lambda b: (b, 0, 0)),
            pl.BlockSpec((1, N, 3), lambda b: (b, 0, 0)),
        ],
        out_specs=[
            pl.BlockSpec((1, N, 1), lambda b: (b, 0, 0)),
            pl.BlockSpec((1, 1, N), lambda b: (b, 0, 0)),
        ],
        out_shape=[
            jax.ShapeDtypeStruct((B, N, 1), jnp.float32),
            jax.ShapeDtypeStruct((B, 1, N), jnp.float32),
        ],
    )(xyz1, xyz2)
    return jnp.mean(d1) + jnp.mean(d2)
